# Initial kernel scaffold; baseline (speedup 1.0000x reference)
import jax
import jax.numpy as jnp
from jax.experimental import pallas as pl


def kernel(text_emb, image_emb, keys, W1, b1, W2, b2, W3, b3):
    raise NotImplementedError("write your pallas kernel here")


# speedup vs baseline: 2.4199x; 2.4199x over previous
import functools

import jax
import jax.numpy as jnp
from jax import lax
from jax.experimental import pallas as pl
from jax.experimental.pallas import tpu as pltpu
from jax.experimental.pallas import tpu_sc as plsc

K_TOP = 3
_INT_BIG = 2**31 - 1


def _mlp_body(t_ref, im_ref, w1_ref, b1_ref, w2_ref, b2_ref, w3_ref, b3_ref,
              out_ref):
    dt = t_ref.shape[1]
    h = (jnp.dot(t_ref[...], w1_ref[0:dt, :], preferred_element_type=jnp.float32)
         + jnp.dot(im_ref[...], w1_ref[dt:, :], preferred_element_type=jnp.float32)
         + b1_ref[...])
    h = jnp.maximum(h, 0.0)
    h = jnp.dot(h, w2_ref[...], preferred_element_type=jnp.float32) + b2_ref[...]
    h = jnp.maximum(h, 0.0)
    p = jnp.dot(h, w3_ref[...], preferred_element_type=jnp.float32) + b3_ref[...]
    nrm = jnp.sqrt(jnp.sum(p * p, axis=1, keepdims=True))
    out_ref[...] = p / (nrm + 1e-12)


def _project(text_emb, image_emb, W1, b1, W2, b2, W3, b3):
    B = text_emb.shape[0]
    dout = W3.shape[1]
    return pl.pallas_call(
        _mlp_body,
        out_shape=jax.ShapeDtypeStruct((B, dout), jnp.float32),
    )(text_emb, image_emb, W1, b1.reshape(1, -1), W2, b2.reshape(1, -1),
      W3, b3.reshape(1, -1))


def _topk_body(nkeys, tile, nt, proj_ref, keys_ref, d_ref, i_ref,
               v0, v1, v2, j0, j1, j2):
    pid = pl.program_id(0)
    B = proj_ref.shape[0]

    @pl.when(pid == 0)
    def _init():
        neg = jnp.full((B, 1), -jnp.inf, jnp.float32)
        v0[...] = neg
        v1[...] = neg
        v2[...] = neg
        zero = jnp.zeros((B, 1), jnp.int32)
        j0[...] = zero
        j1[...] = zero
        j2[...] = zero

    keys_t = keys_ref[...]
    ss = jnp.sum(keys_t * keys_t, axis=1, keepdims=True)
    inv_ok = 1.0 / (jnp.sqrt(ss) + 1e-12)
    row_id = lax.broadcasted_iota(jnp.int32, (tile, 1), 0) + pid * tile
    ks = jnp.where(row_id < nkeys, keys_t * inv_ok, 0.0)

    sim = lax.dot_general(proj_ref[...], ks, (((1,), (1,)), ((), ())),
                          preferred_element_type=jnp.float32)
    gidx = lax.broadcasted_iota(jnp.int32, (B, tile), 1) + pid * tile

    def insert(m, ix):
        c0 = m > v0[...]
        cm = jnp.where(c0, v0[...], m)
        ci = jnp.where(c0, j0[...], ix)
        v0[...] = jnp.where(c0, m, v0[...])
        j0[...] = jnp.where(c0, ix, j0[...])
        c1 = cm > v1[...]
        cm2 = jnp.where(c1, v1[...], cm)
        ci2 = jnp.where(c1, j1[...], ci)
        v1[...] = jnp.where(c1, cm, v1[...])
        j1[...] = jnp.where(c1, ci, j1[...])
        c2 = cm2 > v2[...]
        v2[...] = jnp.where(c2, cm2, v2[...])
        j2[...] = jnp.where(c2, ci2, j2[...])

    for r in range(K_TOP):
        m = jnp.max(sim, axis=1, keepdims=True)
        ix = jnp.min(jnp.where(sim == m, gidx, _INT_BIG), axis=1, keepdims=True)
        if r < K_TOP - 1:
            sim = jnp.where(gidx == ix, -jnp.inf, sim)
        insert(m, ix)

    @pl.when(pid == nt - 1)
    def _flush():
        d_ref[...] = jnp.concatenate([v0[...], v1[...], v2[...]], axis=1)
        i_ref[...] = jnp.concatenate([j0[...], j1[...], j2[...]], axis=1)


def _topk(proj, keys, tile=2048):
    B, dk = proj.shape
    nkeys = keys.shape[0]
    nt = pl.cdiv(nkeys, tile)
    body = functools.partial(_topk_body, nkeys, tile, nt)
    return pl.pallas_call(
        body,
        grid=(nt,),
        in_specs=[
            pl.BlockSpec((B, dk), lambda i: (0, 0)),
            pl.BlockSpec((tile, dk), lambda i: (i, 0)),
        ],
        out_specs=[
            pl.BlockSpec((B, K_TOP), lambda i: (0, 0)),
            pl.BlockSpec((B, K_TOP), lambda i: (0, 0)),
        ],
        out_shape=[
            jax.ShapeDtypeStruct((B, K_TOP), jnp.float32),
            jax.ShapeDtypeStruct((B, K_TOP), jnp.int32),
        ],
        scratch_shapes=[
            pltpu.VMEM((B, 1), jnp.float32),
            pltpu.VMEM((B, 1), jnp.float32),
            pltpu.VMEM((B, 1), jnp.float32),
            pltpu.VMEM((B, 1), jnp.int32),
            pltpu.VMEM((B, 1), jnp.int32),
            pltpu.VMEM((B, 1), jnp.int32),
        ],
    )(proj, keys)


def _sc_gather(keys, idx_flat):
    nkeys, dk = keys.shape
    nb = idx_flat.shape[0]
    info = plsc.get_sparse_core_info()
    nw = info.num_cores * info.num_subcores
    b_per_w = nb // nw
    nchunk = dk // info.num_lanes
    L = info.num_lanes
    mesh = plsc.VectorSubcoreMesh(core_axis_name="c", subcore_axis_name="s")

    @functools.partial(
        pl.kernel,
        mesh=mesh,
        out_type=jax.ShapeDtypeStruct((nb, dk), jnp.float32),
        scratch_types=[
            pltpu.VMEM((b_per_w,), jnp.int32),
            pltpu.VMEM((b_per_w, dk), jnp.float32),
            pltpu.SemaphoreType.DMA,
        ],
        compiler_params=pltpu.CompilerParams(needs_layout_passes=False),
    )
    def gather_kernel(keys_hbm, idx_hbm, out_hbm, idx_v, rows_v, sem):
        wid = lax.axis_index("s") * info.num_cores + lax.axis_index("c")
        base = wid * b_per_w
        pltpu.sync_copy(idx_hbm.at[pl.ds(base, b_per_w)], idx_v)
        pltpu.async_copy(keys_hbm.at[idx_v], rows_v, sem).wait()

        def row_body(r, carry):
            ssv = jnp.zeros((L,), jnp.float32)
            for c in range(nchunk):
                x = rows_v[r, pl.ds(c * L, L)]
                ssv = ssv + x * x
            tot = plsc.cummax(lax.rev(plsc.cumsum(ssv), (0,)))
            ib = lax.bitcast_convert_type(tot, jnp.int32)
            ib = 0x5F3759DF - lax.shift_right_arithmetic(ib, 1)
            y = lax.bitcast_convert_type(ib, jnp.float32)
            for _ in range(3):
                y = y * (1.5 - 0.5 * tot * y * y)
            for c in range(nchunk):
                rows_v[r, pl.ds(c * L, L)] = rows_v[r, pl.ds(c * L, L)] * y
            return carry

        lax.fori_loop(0, b_per_w, row_body, 0)
        pltpu.sync_copy(rows_v, out_hbm.at[pl.ds(base, b_per_w)])

    return gather_kernel(keys, idx_flat)


def kernel(text_emb, image_emb, keys, W1, b1, W2, b2, W3, b3):
    B = text_emb.shape[0]
    dk = keys.shape[1]
    proj = _project(text_emb, image_emb, W1, b1, W2, b2, W3, b3)
    D, I = _topk(proj, keys)
    flat = _sc_gather(keys, I.reshape(-1))
    return flat.reshape(B, K_TOP, dk), D


# speedup vs baseline: 3.8574x; 1.5940x over previous
import functools

import jax
import jax.numpy as jnp
from jax import lax
from jax.experimental import pallas as pl
from jax.experimental.pallas import tpu as pltpu
from jax.experimental.pallas import tpu_sc as plsc

K_TOP = 3
_INT_BIG = 2**31 - 1


def _mlp_body(t_ref, im_ref, w1_ref, b1_ref, w2_ref, b2_ref, w3_ref, b3_ref,
              out_ref):
    dt = t_ref.shape[1]
    h = (jnp.dot(t_ref[...], w1_ref[0:dt, :], preferred_element_type=jnp.float32)
         + jnp.dot(im_ref[...], w1_ref[dt:, :], preferred_element_type=jnp.float32)
         + b1_ref[...])
    h = jnp.maximum(h, 0.0)
    h = jnp.dot(h, w2_ref[...], preferred_element_type=jnp.float32) + b2_ref[...]
    h = jnp.maximum(h, 0.0)
    p = jnp.dot(h, w3_ref[...], preferred_element_type=jnp.float32) + b3_ref[...]
    nrm = jnp.sqrt(jnp.sum(p * p, axis=1, keepdims=True))
    out_ref[...] = p / (nrm + 1e-12)


def _project(text_emb, image_emb, W1, b1, W2, b2, W3, b3):
    B = text_emb.shape[0]
    dout = W3.shape[1]
    return pl.pallas_call(
        _mlp_body,
        out_shape=jax.ShapeDtypeStruct((B, dout), jnp.float32),
    )(text_emb, image_emb, W1, b1.reshape(1, -1), W2, b2.reshape(1, -1),
      W3, b3.reshape(1, -1))


def _topk_body(nkeys, tile, nt, proj_ref, keys_ref, d_ref, i_ref,
               v0, v1, v2, j0, j1, j2):
    pid = pl.program_id(0)
    B = proj_ref.shape[0]

    @pl.when(pid == 0)
    def _init():
        neg = jnp.full((B, 128), -jnp.inf, jnp.float32)
        v0[...] = neg
        v1[...] = neg
        v2[...] = neg
        zero = jnp.zeros((B, 128), jnp.int32)
        j0[...] = zero
        j1[...] = zero
        j2[...] = zero

    keys_t = keys_ref[...]
    ss = jnp.sum(keys_t * keys_t, axis=1, keepdims=True)
    inv_ok = 1.0 / (jnp.sqrt(ss) + 1e-12)
    row_id = lax.broadcasted_iota(jnp.int32, (tile, 1), 0) + pid * tile
    ks = jnp.where(row_id < nkeys, keys_t * inv_ok, 0.0)

    sim = lax.dot_general(proj_ref[...], ks, (((1,), (1,)), ((), ())),
                          preferred_element_type=jnp.float32)

    a0, a1, a2 = v0[...], v1[...], v2[...]
    b0, b1, b2 = j0[...], j1[...], j2[...]
    nchunk = tile // 128
    for c in range(nchunk):
        x = sim[:, c * 128:(c + 1) * 128]
        cid = jnp.full((B, 128), pid * nchunk + c, jnp.int32)
        c0 = x > a0
        cv = jnp.minimum(a0, x)
        ci = jnp.where(c0, b0, cid)
        a0 = jnp.maximum(a0, x)
        b0 = jnp.where(c0, cid, b0)
        c1 = cv > a1
        cv2 = jnp.minimum(a1, cv)
        ci2 = jnp.where(c1, b1, ci)
        a1 = jnp.maximum(a1, cv)
        b1 = jnp.where(c1, ci, b1)
        c2 = cv2 > a2
        a2 = jnp.maximum(a2, cv2)
        b2 = jnp.where(c2, ci2, b2)
    v0[...], v1[...], v2[...] = a0, a1, a2
    j0[...], j1[...], j2[...] = b0, b1, b2

    @pl.when(pid == nt - 1)
    def _flush():
        vs = jnp.concatenate([a0, a1, a2], axis=1)
        lane = lax.broadcasted_iota(jnp.int32, (B, 384), 1) % 128
        gix = jnp.concatenate([b0, b1, b2], axis=1) * 128 + lane
        outs_v, outs_i = [], []
        v, ix = vs, gix
        for r in range(K_TOP):
            m = jnp.max(v, axis=1, keepdims=True)
            jx = jnp.min(jnp.where(v == m, ix, _INT_BIG), axis=1,
                         keepdims=True)
            if r < K_TOP - 1:
                v = jnp.where(ix == jx, -jnp.inf, v)
            outs_v.append(m)
            outs_i.append(jx)
        d_ref[...] = jnp.concatenate(outs_v, axis=1)
        i_ref[...] = jnp.concatenate(outs_i, axis=1)


def _topk(proj, keys, tile=2048):
    B, dk = proj.shape
    nkeys = keys.shape[0]
    nt = pl.cdiv(nkeys, tile)
    body = functools.partial(_topk_body, nkeys, tile, nt)
    return pl.pallas_call(
        body,
        grid=(nt,),
        in_specs=[
            pl.BlockSpec((B, dk), lambda i: (0, 0)),
            pl.BlockSpec((tile, dk), lambda i: (i, 0)),
        ],
        out_specs=[
            pl.BlockSpec((B, K_TOP), lambda i: (0, 0)),
            pl.BlockSpec((B, K_TOP), lambda i: (0, 0)),
        ],
        out_shape=[
            jax.ShapeDtypeStruct((B, K_TOP), jnp.float32),
            jax.ShapeDtypeStruct((B, K_TOP), jnp.int32),
        ],
        scratch_shapes=[
            pltpu.VMEM((B, 128), jnp.float32),
            pltpu.VMEM((B, 128), jnp.float32),
            pltpu.VMEM((B, 128), jnp.float32),
            pltpu.VMEM((B, 128), jnp.int32),
            pltpu.VMEM((B, 128), jnp.int32),
            pltpu.VMEM((B, 128), jnp.int32),
        ],
    )(proj, keys)


def _sc_gather(keys, idx_flat):
    nkeys, dk = keys.shape
    nb = idx_flat.shape[0]
    info = plsc.get_sparse_core_info()
    nw = info.num_cores * info.num_subcores
    b_per_w = nb // nw
    nchunk = dk // info.num_lanes
    L = info.num_lanes
    mesh = plsc.VectorSubcoreMesh(core_axis_name="c", subcore_axis_name="s")

    @functools.partial(
        pl.kernel,
        mesh=mesh,
        out_type=jax.ShapeDtypeStruct((nb, dk), jnp.float32),
        scratch_types=[
            pltpu.VMEM((b_per_w,), jnp.int32),
            pltpu.VMEM((b_per_w, dk), jnp.float32),
            pltpu.SemaphoreType.DMA,
        ],
        compiler_params=pltpu.CompilerParams(needs_layout_passes=False),
    )
    def gather_kernel(keys_hbm, idx_hbm, out_hbm, idx_v, rows_v, sem):
        wid = lax.axis_index("s") * info.num_cores + lax.axis_index("c")
        base = wid * b_per_w
        pltpu.sync_copy(idx_hbm.at[pl.ds(base, b_per_w)], idx_v)
        pltpu.async_copy(keys_hbm.at[idx_v], rows_v, sem).wait()

        def row_body(r, carry):
            ssv = jnp.zeros((L,), jnp.float32)
            for c in range(nchunk):
                x = rows_v[r, pl.ds(c * L, L)]
                ssv = ssv + x * x
            tot = plsc.cummax(lax.rev(plsc.cumsum(ssv), (0,)))
            ib = lax.bitcast_convert_type(tot, jnp.int32)
            ib = 0x5F3759DF - lax.shift_right_arithmetic(ib, 1)
            y = lax.bitcast_convert_type(ib, jnp.float32)
            for _ in range(3):
                y = y * (1.5 - 0.5 * tot * y * y)
            for c in range(nchunk):
                rows_v[r, pl.ds(c * L, L)] = rows_v[r, pl.ds(c * L, L)] * y
            return carry

        lax.fori_loop(0, b_per_w, row_body, 0)
        pltpu.sync_copy(rows_v, out_hbm.at[pl.ds(base, b_per_w)])

    return gather_kernel(keys, idx_flat)


def kernel(text_emb, image_emb, keys, W1, b1, W2, b2, W3, b3):
    B = text_emb.shape[0]
    dk = keys.shape[1]
    proj = _project(text_emb, image_emb, W1, b1, W2, b2, W3, b3)
    D, I = _topk(proj, keys)
    flat = _sc_gather(keys, I.reshape(-1))
    return flat.reshape(B, K_TOP, dk), D
